# baseline (speedup 1.0000x reference)
import jax
import jax.numpy as jnp
from jax import lax
from jax.experimental import pallas as pl
from jax.experimental.pallas import tpu as pltpu
from jax.experimental.pallas import tpu_sc as plsc

N = 10000
NP = 10240
E = 320000
NC = 2
NS = 16
NW = NC * NS
CH = 128
NCHK = 160
NCHKA = 80
ETR = E // NS
ETB = NCHK * CH
F_IN = 128
F_H = 64
FHH = F_H // 2
RPS = NP // NS
R = 1280

_f32 = jnp.float32
_bf16 = jnp.bfloat16
_mesh = plsc.VectorSubcoreMesh(core_axis_name="c", subcore_axis_name="s")


def _deg_body(dst_hbm, w_hbm, out_hbm, dstb, wb, zb, dacc):
    c = lax.axis_index("c")
    s = lax.axis_index("s")
    pltpu.sync_copy(dst_hbm.at[s, pl.ds(c * (NCHK // 2), NCHK // 2)], dstb)
    pltpu.sync_copy(w_hbm.at[s, pl.ds(c * (NCHK // 2), NCHK // 2)], wb)

    zero16 = jnp.zeros((16,), _f32)

    def zloop(r, carry):
        zb[pl.ds(r * 16, 16)] = zero16
        return carry

    lax.fori_loop(0, RPS // 16, zloop, 0)
    base = s * RPS
    pltpu.sync_copy(zb, dacc.at[pl.ds(base, RPS)])
    plsc.subcore_barrier()

    def chunk(j, carry):
        pltpu.sync_copy(wb.at[j], dacc.at[dstb.at[j]], add=True)
        return carry

    lax.fori_loop(0, NCHK // 2, chunk, 0)
    plsc.subcore_barrier()
    pltpu.sync_copy(dacc.at[pl.ds(base, RPS)], out_hbm.at[c, pl.ds(base, RPS)])


_deg_call = pl.kernel(
    _deg_body,
    out_type=jax.ShapeDtypeStruct((NC, NP), _f32),
    mesh=_mesh,
    scratch_types=[
        pltpu.VMEM((NCHK // 2, CH), jnp.int32),
        pltpu.VMEM((NCHK // 2, CH), _f32),
        pltpu.VMEM((RPS,), _f32),
        pltpu.VMEM_SHARED((NP,), _f32),
    ],
)


def _agg_body(hb_hbm, src_hbm, dst_hbm, w_hbm, dinv_hbm, out_hbm,
              srcb, dstb, wb, dinvb, rows0, rows1, rows2, rows3,
              msg0, msg1, msg2, msg3, acc,
              gs0, gs1, gs2, gs3, ss0, ss1, ss2, ss3):
    c = lax.axis_index("c")
    s = lax.axis_index("s")
    wid = c * NS + s
    pltpu.sync_copy(src_hbm.at[wid], srcb)
    pltpu.sync_copy(dst_hbm.at[wid], dstb)
    pltpu.sync_copy(w_hbm.at[wid], wb)
    pltpu.sync_copy(dinv_hbm, dinvb)

    zero16 = jnp.zeros((16,), _f32)

    def zloop(r, carry):
        for k in range(F_H // 16):
            msg0[r, pl.ds(k * 16, 16)] = zero16
        return carry

    lax.fori_loop(0, CH, zloop, 0)
    base = s * RPS
    for t in range(RPS // CH):
        pltpu.sync_copy(msg0, acc.at[pl.ds(base + t * CH, CH)])
    plsc.subcore_barrier()

    def _compute(rows, msg, j):
        def group(g, carry):
            sl = pl.ds(g * 16, 16)
            norm = (plsc.load_gather(dinvb, [srcb[j, sl]]) * wb[j, sl]
                    * plsc.load_gather(dinvb, [dstb[j, sl]]))
            for l in range(16):
                scale = jnp.full((16,), norm[l], _f32)
                r = g * 16 + l
                for k in range(F_H // 32):
                    xb = rows[r, pl.ds(k * 32, 32)]
                    a, b2 = plsc.unpack(xb,
                                        format=plsc.PackFormat.INTERLEAVED)
                    msg[r, pl.ds(k * 32, 16)] = a * scale
                    msg[r, pl.ds(k * 32 + 16, 16)] = b2 * scale
            return carry

        lax.fori_loop(0, CH // 16, group, 0, unroll=4)

    rbufs = (rows0, rows1, rows2, rows3)
    mbufs = (msg0, msg1, msg2, msg3)
    gsems = (gs0, gs1, gs2, gs3)
    ssems = (ss0, ss1, ss2, ss3)

    def block(b, carry):
        jj = b * 4
        gds = [pltpu.async_copy(hb_hbm.at[srcb.at[jj + q]], rbufs[q],
                                gsems[q]) for q in range(4)]
        sds = []
        for q in range(4):
            gds[q].wait()
            _compute(rbufs[q], mbufs[q], jj + q)
            sds.append(pltpu.async_copy(mbufs[q], acc.at[dstb.at[jj + q]],
                                        ssems[q], add=True))
        for q in range(4):
            sds[q].wait()
        return carry

    lax.fori_loop(0, NCHKA // 4, block, 0)
    plsc.subcore_barrier()
    pltpu.sync_copy(acc.at[pl.ds(base, RPS)],
                    out_hbm.at[c, pl.ds(base, RPS)])


_agg_call = pl.kernel(
    _agg_body,
    out_type=jax.ShapeDtypeStruct((NC, NP, F_H), _f32),
    mesh=_mesh,
    compiler_params=pltpu.CompilerParams(needs_layout_passes=False,
                                         use_tc_tiling_on_sc=False),
    scratch_types=[
        pltpu.VMEM((NCHKA, CH), jnp.int32),
        pltpu.VMEM((NCHKA, CH), jnp.int32),
        pltpu.VMEM((NCHKA, CH), _f32),
        pltpu.VMEM((NP,), _f32),
        pltpu.VMEM((CH, F_H), _bf16),
        pltpu.VMEM((CH, F_H), _bf16),
        pltpu.VMEM((CH, F_H), _bf16),
        pltpu.VMEM((CH, F_H), _bf16),
        pltpu.VMEM((CH, F_H), _f32),
        pltpu.VMEM((CH, F_H), _f32),
        pltpu.VMEM((CH, F_H), _f32),
        pltpu.VMEM((CH, F_H), _f32),
        pltpu.VMEM_SHARED((NP, F_H), _f32),
        pltpu.SemaphoreType.DMA,
        pltpu.SemaphoreType.DMA,
        pltpu.SemaphoreType.DMA,
        pltpu.SemaphoreType.DMA,
        pltpu.SemaphoreType.DMA,
        pltpu.SemaphoreType.DMA,
        pltpu.SemaphoreType.DMA,
        pltpu.SemaphoreType.DMA,
    ],
)


def _permuted_bf16(h):
    r = h.shape[0]
    hb = h.astype(_bf16).reshape(r, F_H // 32, 2, 16)
    return hb.swapaxes(2, 3).reshape(r, F_H)


def _tc1_body(pt_ref, x_ref, w1_ref, h1_ref, h1b_ref, dinv_ref,
              invdeg_ref):
    p = pt_ref[...]
    deg = p[:, 0:1] + p[:, 1:2] + 1.0
    invdeg_ref[...] = 1.0 / deg
    dinv_ref[...] = lax.rsqrt(deg)
    h1 = jnp.dot(x_ref[...], w1_ref[...], preferred_element_type=_f32)
    h1_ref[...] = h1
    h1b_ref[...] = _permuted_bf16(h1)


_tc1 = pl.pallas_call(
    _tc1_body,
    grid=(NP // R,),
    in_specs=[
        pl.BlockSpec((R, 2), lambda i: (i, 0)),
        pl.BlockSpec((R, F_IN), lambda i: (i, 0)),
        pl.BlockSpec((F_IN, F_H), lambda i: (0, 0)),
    ],
    out_specs=[
        pl.BlockSpec((R, F_H), lambda i: (i, 0)),
        pl.BlockSpec((R, F_H), lambda i: (i, 0)),
        pl.BlockSpec((R, 1), lambda i: (i, 0)),
        pl.BlockSpec((R, 1), lambda i: (i, 0)),
    ],
    out_shape=[
        jax.ShapeDtypeStruct((NP, F_H), _f32),
        jax.ShapeDtypeStruct((NP, F_H), _bf16),
        jax.ShapeDtypeStruct((NP, 1), _f32),
        jax.ShapeDtypeStruct((NP, 1), _f32),
    ],
)


def _tc2_body(s_ref, h_ref, invdeg_ref, b_ref, w_ref, h2_ref, h2b_ref):
    sarr = s_ref[...]
    agg = sarr[0] + sarr[1]
    z = agg + h_ref[...] * invdeg_ref[...] + b_ref[...]
    z = jnp.maximum(z, 0.0)
    h2 = jnp.dot(z, w_ref[...], preferred_element_type=_f32)
    h2_ref[...] = h2
    h2b_ref[...] = _permuted_bf16(h2)


_tc2 = pl.pallas_call(
    _tc2_body,
    grid=(NP // R,),
    in_specs=[
        pl.BlockSpec((NC, R, F_H), lambda i: (0, i, 0)),
        pl.BlockSpec((R, F_H), lambda i: (i, 0)),
        pl.BlockSpec((R, 1), lambda i: (i, 0)),
        pl.BlockSpec((1, F_H), lambda i: (0, 0)),
        pl.BlockSpec((F_H, F_H), lambda i: (0, 0)),
    ],
    out_specs=[
        pl.BlockSpec((R, F_H), lambda i: (i, 0)),
        pl.BlockSpec((R, F_H), lambda i: (i, 0)),
    ],
    out_shape=[
        jax.ShapeDtypeStruct((NP, F_H), _f32),
        jax.ShapeDtypeStruct((NP, F_H), _bf16),
    ],
)


def _head_body(s_ref, h_ref, invdeg_ref, b_ref, wl_ref, bl_ref, out_ref):
    sarr = s_ref[...]
    agg = sarr[0] + sarr[1]
    z = agg + h_ref[...] * invdeg_ref[...] + b_ref[...]
    z = jnp.maximum(z, 0.0)
    out_ref[...] = jnp.dot(z, wl_ref[...],
                           preferred_element_type=_f32) + bl_ref[...]


_tc3 = pl.pallas_call(
    _head_body,
    grid=(NP // R,),
    in_specs=[
        pl.BlockSpec((NC, R, F_H), lambda i: (0, i, 0)),
        pl.BlockSpec((R, F_H), lambda i: (i, 0)),
        pl.BlockSpec((R, 1), lambda i: (i, 0)),
        pl.BlockSpec((1, F_H), lambda i: (0, 0)),
        pl.BlockSpec((F_H, 1), lambda i: (0, 0)),
        pl.BlockSpec((1, 1), lambda i: (0, 0)),
    ],
    out_specs=pl.BlockSpec((R, 1), lambda i: (i, 0)),
    out_shape=jax.ShapeDtypeStruct((NP, 1), _f32),
)


def kernel(x, edge_index, edge_weight, W1, b1, W2, b2, Wl, bl):
    def shard(a, nway):
        per = E // nway
        perb = (ETB * NS) // nway
        a2 = a.reshape(nway, per)
        z = jnp.zeros((nway, perb - per), a.dtype)
        return jnp.concatenate([a2, z], axis=1).reshape(nway, perb // CH, CH)

    dst16 = shard(edge_index[1], NS)
    w16 = shard(edge_weight, NS)
    src_p = shard(edge_index[0], NW)
    dst_p = shard(edge_index[1], NW)
    w_p = shard(edge_weight, NW)
    x_p = jnp.pad(x, ((0, NP - N), (0, 0)))

    deg_parts = _deg_call(dst16, w16)
    h1, h1b, dinv_col, invdeg_col = _tc1(deg_parts.T, x_p, W1)
    dinv = dinv_col.reshape(NP)
    s1 = _agg_call(h1b, src_p, dst_p, w_p, dinv)
    h2, h2b = _tc2(s1, h1, invdeg_col, b1.reshape(1, F_H), W2)
    s2 = _agg_call(h2b, src_p, dst_p, w_p, dinv)
    out_col = _tc3(s2, h2, invdeg_col, b2.reshape(1, F_H),
                   Wl, bl.reshape(1, 1))
    return out_col[:N, 0]


# speedup vs baseline: 1.2483x; 1.2483x over previous
import jax
import jax.numpy as jnp
from jax import lax
from jax.experimental import pallas as pl
from jax.experimental.pallas import tpu as pltpu
from jax.experimental.pallas import tpu_sc as plsc

N = 10000
NP = 10240
E = 320000
NC = 2
NS = 16
NW = NC * NS
CH = 128
NCHK = 160
NCHKA = 80
F_IN = 128
F_H = 64
RPS = NP // NS
R = 1280

_f32 = jnp.float32
_mesh = plsc.VectorSubcoreMesh(core_axis_name="c", subcore_axis_name="s")


def _deg_body(dst_hbm, w_hbm, out_hbm, dstb, wb, zb, dacc):
    c = lax.axis_index("c")
    s = lax.axis_index("s")
    pltpu.sync_copy(dst_hbm.at[s, pl.ds(c * (NCHK // 2), NCHK // 2)], dstb)
    pltpu.sync_copy(w_hbm.at[s, pl.ds(c * (NCHK // 2), NCHK // 2)], wb)

    zero16 = jnp.zeros((16,), _f32)

    def zloop(r, carry):
        zb[pl.ds(r * 16, 16)] = zero16
        return carry

    lax.fori_loop(0, RPS // 16, zloop, 0)
    base = s * RPS
    pltpu.sync_copy(zb, dacc.at[pl.ds(base, RPS)])
    plsc.subcore_barrier()

    def chunk(j, carry):
        pltpu.sync_copy(wb.at[j], dacc.at[dstb.at[j]], add=True)
        return carry

    lax.fori_loop(0, NCHK // 2, chunk, 0)
    plsc.subcore_barrier()
    pltpu.sync_copy(dacc.at[pl.ds(base, RPS)], out_hbm.at[c, pl.ds(base, RPS)])


_deg_call = pl.kernel(
    _deg_body,
    out_type=jax.ShapeDtypeStruct((NC, NP), _f32),
    mesh=_mesh,
    scratch_types=[
        pltpu.VMEM((NCHK // 2, CH), jnp.int32),
        pltpu.VMEM((NCHK // 2, CH), _f32),
        pltpu.VMEM((RPS,), _f32),
        pltpu.VMEM_SHARED((NP,), _f32),
    ],
)


def _agg_body(g_hbm, src_hbm, dst_hbm, w_hbm, out_hbm,
              srcb, dstb, wb, rows0, rows1, rows2, rows3, acc,
              gs0, gs1, gs2, gs3, ss0, ss1, ss2, ss3):
    c = lax.axis_index("c")
    s = lax.axis_index("s")
    wid = c * NS + s
    pltpu.sync_copy(src_hbm.at[wid], srcb)
    pltpu.sync_copy(dst_hbm.at[wid], dstb)
    pltpu.sync_copy(w_hbm.at[wid], wb)

    zero16 = jnp.zeros((16,), _f32)

    def zloop(r, carry):
        for k in range(F_H // 16):
            rows0[r, pl.ds(k * 16, 16)] = zero16
        return carry

    lax.fori_loop(0, CH, zloop, 0)
    base = s * RPS
    for t in range(RPS // CH):
        pltpu.sync_copy(rows0, acc.at[pl.ds(base + t * CH, CH)])
    plsc.subcore_barrier()

    def _compute(rows, j):
        def group(g, carry):
            wv = wb[j, pl.ds(g * 16, 16)]
            for l in range(16):
                scale = jnp.full((16,), wv[l], _f32)
                r = g * 16 + l
                for k in range(F_H // 16):
                    fsl = pl.ds(k * 16, 16)
                    rows[r, fsl] = rows[r, fsl] * scale
            return carry

        lax.fori_loop(0, CH // 16, group, 0, unroll=4)

    rbufs = (rows0, rows1, rows2, rows3)
    gsems = (gs0, gs1, gs2, gs3)
    ssems = (ss0, ss1, ss2, ss3)

    def block(b, carry):
        jj = b * 4
        gds = [pltpu.async_copy(g_hbm.at[srcb.at[jj + q]], rbufs[q],
                                gsems[q]) for q in range(4)]
        sds = []
        for q in range(4):
            gds[q].wait()
            _compute(rbufs[q], jj + q)
            sds.append(pltpu.async_copy(rbufs[q], acc.at[dstb.at[jj + q]],
                                        ssems[q], add=True))
        for q in range(4):
            sds[q].wait()
        return carry

    lax.fori_loop(0, NCHKA // 4, block, 0)
    plsc.subcore_barrier()
    pltpu.sync_copy(acc.at[pl.ds(base, RPS)],
                    out_hbm.at[c, pl.ds(base, RPS)])


_agg_call = pl.kernel(
    _agg_body,
    out_type=jax.ShapeDtypeStruct((NC, NP, F_H), _f32),
    mesh=_mesh,
    compiler_params=pltpu.CompilerParams(needs_layout_passes=False,
                                         use_tc_tiling_on_sc=False),
    scratch_types=[
        pltpu.VMEM((NCHKA, CH), jnp.int32),
        pltpu.VMEM((NCHKA, CH), jnp.int32),
        pltpu.VMEM((NCHKA, CH), _f32),
        pltpu.VMEM((CH, F_H), _f32),
        pltpu.VMEM((CH, F_H), _f32),
        pltpu.VMEM((CH, F_H), _f32),
        pltpu.VMEM((CH, F_H), _f32),
        pltpu.VMEM_SHARED((NP, F_H), _f32),
        pltpu.SemaphoreType.DMA,
        pltpu.SemaphoreType.DMA,
        pltpu.SemaphoreType.DMA,
        pltpu.SemaphoreType.DMA,
        pltpu.SemaphoreType.DMA,
        pltpu.SemaphoreType.DMA,
        pltpu.SemaphoreType.DMA,
        pltpu.SemaphoreType.DMA,
    ],
)


def _mm_body(x_ref, w_ref, out_ref):
    out_ref[...] = jnp.dot(x_ref[...], w_ref[...],
                           preferred_element_type=_f32)


_tc_mm = pl.pallas_call(
    _mm_body,
    grid=(NP // R,),
    in_specs=[
        pl.BlockSpec((R, F_IN), lambda i: (i, 0)),
        pl.BlockSpec((F_IN, F_H), lambda i: (0, 0)),
    ],
    out_specs=pl.BlockSpec((R, F_H), lambda i: (i, 0)),
    out_shape=jax.ShapeDtypeStruct((NP, F_H), _f32),
)


def _scale_body(pt_ref, h_ref, g_ref, dinv_ref):
    p = pt_ref[...]
    deg = p[:, 0:1] + p[:, 1:2] + 1.0
    dinv = lax.rsqrt(deg)
    dinv_ref[...] = dinv
    g_ref[...] = h_ref[...] * dinv


_tc_scale = pl.pallas_call(
    _scale_body,
    grid=(NP // R,),
    in_specs=[
        pl.BlockSpec((R, 2), lambda i: (i, 0)),
        pl.BlockSpec((R, F_H), lambda i: (i, 0)),
    ],
    out_specs=[
        pl.BlockSpec((R, F_H), lambda i: (i, 0)),
        pl.BlockSpec((R, 1), lambda i: (i, 0)),
    ],
    out_shape=[
        jax.ShapeDtypeStruct((NP, F_H), _f32),
        jax.ShapeDtypeStruct((NP, 1), _f32),
    ],
)


def _tc2_body(s_ref, g_ref, dinv_ref, b_ref, w_ref, g2_ref):
    sarr = s_ref[...]
    dinv = dinv_ref[...]
    z = (sarr[0] + sarr[1] + g_ref[...]) * dinv + b_ref[...]
    z = jnp.maximum(z, 0.0)
    h2 = jnp.dot(z, w_ref[...], preferred_element_type=_f32)
    g2_ref[...] = h2 * dinv


_tc2 = pl.pallas_call(
    _tc2_body,
    grid=(NP // R,),
    in_specs=[
        pl.BlockSpec((NC, R, F_H), lambda i: (0, i, 0)),
        pl.BlockSpec((R, F_H), lambda i: (i, 0)),
        pl.BlockSpec((R, 1), lambda i: (i, 0)),
        pl.BlockSpec((1, F_H), lambda i: (0, 0)),
        pl.BlockSpec((F_H, F_H), lambda i: (0, 0)),
    ],
    out_specs=pl.BlockSpec((R, F_H), lambda i: (i, 0)),
    out_shape=jax.ShapeDtypeStruct((NP, F_H), _f32),
)


def _head_body(s_ref, g_ref, dinv_ref, b_ref, wl_ref, bl_ref, out_ref):
    sarr = s_ref[...]
    z = (sarr[0] + sarr[1] + g_ref[...]) * dinv_ref[...] + b_ref[...]
    z = jnp.maximum(z, 0.0)
    out_ref[...] = jnp.dot(z, wl_ref[...],
                           preferred_element_type=_f32) + bl_ref[...]


_tc3 = pl.pallas_call(
    _head_body,
    grid=(NP // R,),
    in_specs=[
        pl.BlockSpec((NC, R, F_H), lambda i: (0, i, 0)),
        pl.BlockSpec((R, F_H), lambda i: (i, 0)),
        pl.BlockSpec((R, 1), lambda i: (i, 0)),
        pl.BlockSpec((1, F_H), lambda i: (0, 0)),
        pl.BlockSpec((F_H, 1), lambda i: (0, 0)),
        pl.BlockSpec((1, 1), lambda i: (0, 0)),
    ],
    out_specs=pl.BlockSpec((R, 1), lambda i: (i, 0)),
    out_shape=jax.ShapeDtypeStruct((NP, 1), _f32),
)


def kernel(x, edge_index, edge_weight, W1, b1, W2, b2, Wl, bl):
    def shard(a, nway):
        per = E // nway
        perb = (NCHK * CH * NS) // nway
        a2 = a.reshape(nway, per)
        z = jnp.zeros((nway, perb - per), a.dtype)
        return jnp.concatenate([a2, z], axis=1).reshape(nway, perb // CH, CH)

    dst16 = shard(edge_index[1], NS)
    w16 = shard(edge_weight, NS)
    src_p = shard(edge_index[0], NW)
    dst_p = shard(edge_index[1], NW)
    w_p = shard(edge_weight, NW)
    x_p = jnp.pad(x, ((0, NP - N), (0, 0)))

    deg_parts = _deg_call(dst16, w16)
    h1 = _tc_mm(x_p, W1)
    g1, dinv_col = _tc_scale(deg_parts.T, h1)
    s1 = _agg_call(g1, src_p, dst_p, w_p)
    g2 = _tc2(s1, g1, dinv_col, b1.reshape(1, F_H), W2)
    s2 = _agg_call(g2, src_p, dst_p, w_p)
    out_col = _tc3(s2, g2, dinv_col, b2.reshape(1, F_H),
                   Wl, bl.reshape(1, 1))
    return out_col[:N, 0]


# speedup vs baseline: 1.2489x; 1.0005x over previous
import jax
import jax.numpy as jnp
from jax import lax
from jax.experimental import pallas as pl
from jax.experimental.pallas import tpu as pltpu
from jax.experimental.pallas import tpu_sc as plsc

N = 10000
NP = 10240
E = 320000
NC = 2
NS = 16
NW = NC * NS
CH = 128
NCHK = 160
NCHKA = 80
F_IN = 128
F_H = 64
RPS = NP // NS
R = 1280

_f32 = jnp.float32
_mesh = plsc.VectorSubcoreMesh(core_axis_name="c", subcore_axis_name="s")


def _deg_body(dst_hbm, w_hbm, out_hbm, dstb, wb, zb, dacc):
    c = lax.axis_index("c")
    s = lax.axis_index("s")
    pltpu.sync_copy(dst_hbm.at[s, pl.ds(c * (NCHK // 2), NCHK // 2)], dstb)
    pltpu.sync_copy(w_hbm.at[s, pl.ds(c * (NCHK // 2), NCHK // 2)], wb)

    zero16 = jnp.zeros((16,), _f32)

    def zloop(r, carry):
        zb[pl.ds(r * 16, 16)] = zero16
        return carry

    lax.fori_loop(0, RPS // 16, zloop, 0)
    base = s * RPS
    pltpu.sync_copy(zb, dacc.at[pl.ds(base, RPS)])
    plsc.subcore_barrier()

    def chunk(j, carry):
        pltpu.sync_copy(wb.at[j], dacc.at[dstb.at[j]], add=True)
        return carry

    lax.fori_loop(0, NCHK // 2, chunk, 0)
    plsc.subcore_barrier()
    pltpu.sync_copy(dacc.at[pl.ds(base, RPS)], out_hbm.at[c, pl.ds(base, RPS)])


_deg_call = pl.kernel(
    _deg_body,
    out_type=jax.ShapeDtypeStruct((NC, NP), _f32),
    mesh=_mesh,
    scratch_types=[
        pltpu.VMEM((NCHK // 2, CH), jnp.int32),
        pltpu.VMEM((NCHK // 2, CH), _f32),
        pltpu.VMEM((RPS,), _f32),
        pltpu.VMEM_SHARED((NP,), _f32),
    ],
)


def _agg_body(g_hbm, src_hbm, dst_hbm, w_hbm, out_hbm,
              srcb, dstb, wb, rows0, rows1, rows2, rows3, acc,
              gs0, gs1, gs2, gs3, ss0, ss1, ss2, ss3):
    c = lax.axis_index("c")
    s = lax.axis_index("s")
    wid = c * NS + s
    pltpu.sync_copy(src_hbm.at[wid], srcb)
    pltpu.sync_copy(dst_hbm.at[wid], dstb)
    pltpu.sync_copy(w_hbm.at[wid], wb)

    zero16 = jnp.zeros((16,), _f32)

    def zloop(r, carry):
        for k in range(F_H // 16):
            rows0[r, pl.ds(k * 16, 16)] = zero16
        return carry

    lax.fori_loop(0, CH, zloop, 0)
    base = s * RPS
    for t in range(RPS // CH):
        pltpu.sync_copy(rows0, acc.at[pl.ds(base + t * CH, CH)])
    plsc.subcore_barrier()

    def _compute(rows, j):
        def group(g, carry):
            wv = wb[j, pl.ds(g * 16, 16)]
            for l in range(16):
                scale = jnp.full((16,), wv[l], _f32)
                r = g * 16 + l
                for k in range(F_H // 16):
                    fsl = pl.ds(k * 16, 16)
                    rows[r, fsl] = rows[r, fsl] * scale
            return carry

        lax.fori_loop(0, CH // 16, group, 0, unroll=4)

    rbufs = (rows0, rows1, rows2, rows3)
    gsems = (gs0, gs1, gs2, gs3)
    ssems = (ss0, ss1, ss2, ss3)

    def block(b, carry):
        jj = b * 4
        gds = [pltpu.async_copy(g_hbm.at[srcb.at[jj + q]], rbufs[q],
                                gsems[q]) for q in range(4)]
        sds = []
        for q in range(4):
            gds[q].wait()
            _compute(rbufs[q], jj + q)
            sds.append(pltpu.async_copy(rbufs[q], acc.at[dstb.at[jj + q]],
                                        ssems[q], add=True))
        for q in range(4):
            sds[q].wait()
        return carry

    lax.fori_loop(0, NCHKA // 4, block, 0)
    plsc.subcore_barrier()
    pltpu.sync_copy(acc.at[pl.ds(base, RPS)],
                    out_hbm.at[c, pl.ds(base, RPS)])


_agg_call = pl.kernel(
    _agg_body,
    out_type=jax.ShapeDtypeStruct((NC, NP, F_H), _f32),
    mesh=_mesh,
    compiler_params=pltpu.CompilerParams(needs_layout_passes=False,
                                         use_tc_tiling_on_sc=False),
    scratch_types=[
        pltpu.VMEM((NCHKA, CH), jnp.int32),
        pltpu.VMEM((NCHKA, CH), jnp.int32),
        pltpu.VMEM((NCHKA, CH), _f32),
        pltpu.VMEM((CH, F_H), _f32),
        pltpu.VMEM((CH, F_H), _f32),
        pltpu.VMEM((CH, F_H), _f32),
        pltpu.VMEM((CH, F_H), _f32),
        pltpu.VMEM_SHARED((NP, F_H), _f32),
        pltpu.SemaphoreType.DMA,
        pltpu.SemaphoreType.DMA,
        pltpu.SemaphoreType.DMA,
        pltpu.SemaphoreType.DMA,
        pltpu.SemaphoreType.DMA,
        pltpu.SemaphoreType.DMA,
        pltpu.SemaphoreType.DMA,
        pltpu.SemaphoreType.DMA,
    ],
)


def _tc1_body(pt_ref, x_ref, w_ref, g_ref, dinv_ref):
    p = pt_ref[...]
    deg = p[:, 0:1] + p[:, 1:2] + 1.0
    dinv = lax.rsqrt(deg)
    dinv_ref[...] = dinv
    h = jnp.dot(x_ref[...], w_ref[...], preferred_element_type=_f32)
    g_ref[...] = h * dinv


_tc1 = pl.pallas_call(
    _tc1_body,
    grid=(NP // R,),
    in_specs=[
        pl.BlockSpec((R, 2), lambda i: (i, 0)),
        pl.BlockSpec((R, F_IN), lambda i: (i, 0)),
        pl.BlockSpec((F_IN, F_H), lambda i: (0, 0)),
    ],
    out_specs=[
        pl.BlockSpec((R, F_H), lambda i: (i, 0)),
        pl.BlockSpec((R, 1), lambda i: (i, 0)),
    ],
    out_shape=[
        jax.ShapeDtypeStruct((NP, F_H), _f32),
        jax.ShapeDtypeStruct((NP, 1), _f32),
    ],
)


def _tc2_body(s_ref, g_ref, dinv_ref, b_ref, w_ref, g2_ref):
    sarr = s_ref[...]
    dinv = dinv_ref[...]
    z = (sarr[0] + sarr[1] + g_ref[...]) * dinv + b_ref[...]
    z = jnp.maximum(z, 0.0)
    h2 = jnp.dot(z, w_ref[...], preferred_element_type=_f32)
    g2_ref[...] = h2 * dinv


_tc2 = pl.pallas_call(
    _tc2_body,
    grid=(NP // R,),
    in_specs=[
        pl.BlockSpec((NC, R, F_H), lambda i: (0, i, 0)),
        pl.BlockSpec((R, F_H), lambda i: (i, 0)),
        pl.BlockSpec((R, 1), lambda i: (i, 0)),
        pl.BlockSpec((1, F_H), lambda i: (0, 0)),
        pl.BlockSpec((F_H, F_H), lambda i: (0, 0)),
    ],
    out_specs=pl.BlockSpec((R, F_H), lambda i: (i, 0)),
    out_shape=jax.ShapeDtypeStruct((NP, F_H), _f32),
)


def _head_body(s_ref, g_ref, dinv_ref, b_ref, wl_ref, bl_ref, out_ref):
    sarr = s_ref[...]
    z = (sarr[0] + sarr[1] + g_ref[...]) * dinv_ref[...] + b_ref[...]
    z = jnp.maximum(z, 0.0)
    out_ref[...] = jnp.dot(z, wl_ref[...],
                           preferred_element_type=_f32) + bl_ref[...]


_tc3 = pl.pallas_call(
    _head_body,
    grid=(NP // R,),
    in_specs=[
        pl.BlockSpec((NC, R, F_H), lambda i: (0, i, 0)),
        pl.BlockSpec((R, F_H), lambda i: (i, 0)),
        pl.BlockSpec((R, 1), lambda i: (i, 0)),
        pl.BlockSpec((1, F_H), lambda i: (0, 0)),
        pl.BlockSpec((F_H, 1), lambda i: (0, 0)),
        pl.BlockSpec((1, 1), lambda i: (0, 0)),
    ],
    out_specs=pl.BlockSpec((R, 1), lambda i: (i, 0)),
    out_shape=jax.ShapeDtypeStruct((NP, 1), _f32),
)


def kernel(x, edge_index, edge_weight, W1, b1, W2, b2, Wl, bl):
    def shard(a, nway):
        per = E // nway
        perb = (NCHK * CH * NS) // nway
        a2 = a.reshape(nway, per)
        z = jnp.zeros((nway, perb - per), a.dtype)
        return jnp.concatenate([a2, z], axis=1).reshape(nway, perb // CH, CH)

    dst16 = shard(edge_index[1], NS)
    w16 = shard(edge_weight, NS)
    src_p = shard(edge_index[0], NW)
    dst_p = shard(edge_index[1], NW)
    w_p = shard(edge_weight, NW)
    x_p = jnp.pad(x, ((0, NP - N), (0, 0)))

    deg_parts = _deg_call(dst16, w16)
    g1, dinv_col = _tc1(deg_parts.T, x_p, W1)
    s1 = _agg_call(g1, src_p, dst_p, w_p)
    g2 = _tc2(s1, g1, dinv_col, b1.reshape(1, F_H), W2)
    s2 = _agg_call(g2, src_p, dst_p, w_p)
    out_col = _tc3(s2, g2, dinv_col, b2.reshape(1, F_H),
                   Wl, bl.reshape(1, 1))
    return out_col[:N, 0]


# speedup vs baseline: 1.2613x; 1.0100x over previous
import jax
import jax.numpy as jnp
from jax import lax
from jax.experimental import pallas as pl
from jax.experimental.pallas import tpu as pltpu
from jax.experimental.pallas import tpu_sc as plsc

N = 10000
NP = 10240
E = 320000
NC = 2
NS = 16
NW = NC * NS
CH = 128
NCHK = 160
NCHKA = 80
F_IN = 128
F_H = 64
RPS = NP // NS
R = 1280

_f32 = jnp.float32
_mesh = plsc.VectorSubcoreMesh(core_axis_name="c", subcore_axis_name="s")


def _deg_body(dst_hbm, w_hbm, out_hbm, dstb, wb, zb, dacc):
    c = lax.axis_index("c")
    s = lax.axis_index("s")
    pltpu.sync_copy(dst_hbm.at[s, pl.ds(c * (NCHK // 2), NCHK // 2)], dstb)
    pltpu.sync_copy(w_hbm.at[s, pl.ds(c * (NCHK // 2), NCHK // 2)], wb)

    zero16 = jnp.zeros((16,), _f32)

    def zloop(r, carry):
        zb[pl.ds(r * 16, 16)] = zero16
        return carry

    lax.fori_loop(0, RPS // 16, zloop, 0)
    base = s * RPS
    pltpu.sync_copy(zb, dacc.at[pl.ds(base, RPS)])
    plsc.subcore_barrier()

    def chunk(j, carry):
        pltpu.sync_copy(wb.at[j], dacc.at[dstb.at[j]], add=True)
        return carry

    lax.fori_loop(0, NCHK // 2, chunk, 0)
    plsc.subcore_barrier()
    pltpu.sync_copy(dacc.at[pl.ds(base, RPS)], out_hbm.at[c, pl.ds(base, RPS)])


_deg_call = pl.kernel(
    _deg_body,
    out_type=jax.ShapeDtypeStruct((NC, NP), _f32),
    mesh=_mesh,
    scratch_types=[
        pltpu.VMEM((NCHK // 2, CH), jnp.int32),
        pltpu.VMEM((NCHK // 2, CH), _f32),
        pltpu.VMEM((RPS,), _f32),
        pltpu.VMEM_SHARED((NP,), _f32),
    ],
)


def _agg_body(g_hbm, src_hbm, dst_hbm, w_hbm, out_hbm,
              srcb, dstb, wb, rows0, rows1, rows2, rows3, acc,
              gs0, gs1, gs2, gs3, ss0, ss1, ss2, ss3):
    c = lax.axis_index("c")
    s = lax.axis_index("s")
    wid = c * NS + s
    sd0 = pltpu.async_copy(src_hbm.at[wid], srcb, gs0)
    sd1 = pltpu.async_copy(dst_hbm.at[wid], dstb, gs1)
    sd2 = pltpu.async_copy(w_hbm.at[wid], wb, gs2)

    zero16 = jnp.zeros((16,), _f32)

    def zloop(r, carry):
        for k in range(F_H // 16):
            rows0[r, pl.ds(k * 16, 16)] = zero16
        return carry

    lax.fori_loop(0, CH, zloop, 0)
    base = s * RPS
    for t in range(RPS // CH):
        pltpu.sync_copy(rows0, acc.at[pl.ds(base + t * CH, CH)])
    sd0.wait()
    sd1.wait()
    sd2.wait()
    plsc.subcore_barrier()

    def _compute(rows, j):
        def group(g, carry):
            wv = wb[j, pl.ds(g * 16, 16)]
            for l in range(16):
                scale = jnp.full((16,), wv[l], _f32)
                r = g * 16 + l
                for k in range(F_H // 16):
                    fsl = pl.ds(k * 16, 16)
                    rows[r, fsl] = rows[r, fsl] * scale
            return carry

        lax.fori_loop(0, CH // 16, group, 0, unroll=4)

    rbufs = (rows0, rows1, rows2, rows3)
    gsems = (gs0, gs1, gs2, gs3)
    ssems = (ss0, ss1, ss2, ss3)

    def block(b, carry):
        jj = b * 4
        gds = [pltpu.async_copy(g_hbm.at[srcb.at[jj + q]], rbufs[q],
                                gsems[q]) for q in range(4)]
        sds = []
        for q in range(4):
            gds[q].wait()
            _compute(rbufs[q], jj + q)
            sds.append(pltpu.async_copy(rbufs[q], acc.at[dstb.at[jj + q]],
                                        ssems[q], add=True))
        for q in range(4):
            sds[q].wait()
        return carry

    lax.fori_loop(0, NCHKA // 4, block, 0)
    plsc.subcore_barrier()
    pltpu.sync_copy(acc.at[pl.ds(base, RPS)],
                    out_hbm.at[c, pl.ds(base, RPS)])


_agg_call = pl.kernel(
    _agg_body,
    out_type=jax.ShapeDtypeStruct((NC, NP, F_H), _f32),
    mesh=_mesh,
    compiler_params=pltpu.CompilerParams(needs_layout_passes=False,
                                         use_tc_tiling_on_sc=False),
    scratch_types=[
        pltpu.VMEM((NCHKA, CH), jnp.int32),
        pltpu.VMEM((NCHKA, CH), jnp.int32),
        pltpu.VMEM((NCHKA, CH), _f32),
        pltpu.VMEM((CH, F_H), _f32),
        pltpu.VMEM((CH, F_H), _f32),
        pltpu.VMEM((CH, F_H), _f32),
        pltpu.VMEM((CH, F_H), _f32),
        pltpu.VMEM_SHARED((NP, F_H), _f32),
        pltpu.SemaphoreType.DMA,
        pltpu.SemaphoreType.DMA,
        pltpu.SemaphoreType.DMA,
        pltpu.SemaphoreType.DMA,
        pltpu.SemaphoreType.DMA,
        pltpu.SemaphoreType.DMA,
        pltpu.SemaphoreType.DMA,
        pltpu.SemaphoreType.DMA,
    ],
)


def _tc1_body(pt_ref, x_ref, w_ref, g_ref, dinv_ref):
    p = pt_ref[...]
    deg = p[:, 0:1] + p[:, 1:2] + 1.0
    dinv = lax.rsqrt(deg)
    dinv_ref[...] = dinv
    h = jnp.dot(x_ref[...], w_ref[...], preferred_element_type=_f32)
    g_ref[...] = h * dinv


_tc1 = pl.pallas_call(
    _tc1_body,
    grid=(NP // R,),
    in_specs=[
        pl.BlockSpec((R, 2), lambda i: (i, 0)),
        pl.BlockSpec((R, F_IN), lambda i: (i, 0)),
        pl.BlockSpec((F_IN, F_H), lambda i: (0, 0)),
    ],
    out_specs=[
        pl.BlockSpec((R, F_H), lambda i: (i, 0)),
        pl.BlockSpec((R, 1), lambda i: (i, 0)),
    ],
    out_shape=[
        jax.ShapeDtypeStruct((NP, F_H), _f32),
        jax.ShapeDtypeStruct((NP, 1), _f32),
    ],
)


def _tc2_body(s_ref, g_ref, dinv_ref, b_ref, w_ref, g2_ref):
    sarr = s_ref[...]
    dinv = dinv_ref[...]
    z = (sarr[0] + sarr[1] + g_ref[...]) * dinv + b_ref[...]
    z = jnp.maximum(z, 0.0)
    h2 = jnp.dot(z, w_ref[...], preferred_element_type=_f32)
    g2_ref[...] = h2 * dinv


_tc2 = pl.pallas_call(
    _tc2_body,
    grid=(NP // R,),
    in_specs=[
        pl.BlockSpec((NC, R, F_H), lambda i: (0, i, 0)),
        pl.BlockSpec((R, F_H), lambda i: (i, 0)),
        pl.BlockSpec((R, 1), lambda i: (i, 0)),
        pl.BlockSpec((1, F_H), lambda i: (0, 0)),
        pl.BlockSpec((F_H, F_H), lambda i: (0, 0)),
    ],
    out_specs=pl.BlockSpec((R, F_H), lambda i: (i, 0)),
    out_shape=jax.ShapeDtypeStruct((NP, F_H), _f32),
)


def _head_body(s_ref, g_ref, dinv_ref, b_ref, wl_ref, bl_ref, out_ref):
    sarr = s_ref[...]
    z = (sarr[0] + sarr[1] + g_ref[...]) * dinv_ref[...] + b_ref[...]
    z = jnp.maximum(z, 0.0)
    out_ref[...] = jnp.dot(z, wl_ref[...],
                           preferred_element_type=_f32) + bl_ref[...]


_tc3 = pl.pallas_call(
    _head_body,
    grid=(NP // R,),
    in_specs=[
        pl.BlockSpec((NC, R, F_H), lambda i: (0, i, 0)),
        pl.BlockSpec((R, F_H), lambda i: (i, 0)),
        pl.BlockSpec((R, 1), lambda i: (i, 0)),
        pl.BlockSpec((1, F_H), lambda i: (0, 0)),
        pl.BlockSpec((F_H, 1), lambda i: (0, 0)),
        pl.BlockSpec((1, 1), lambda i: (0, 0)),
    ],
    out_specs=pl.BlockSpec((R, 1), lambda i: (i, 0)),
    out_shape=jax.ShapeDtypeStruct((NP, 1), _f32),
)


def kernel(x, edge_index, edge_weight, W1, b1, W2, b2, Wl, bl):
    def shard(a, nway):
        per = E // nway
        perb = (NCHK * CH * NS) // nway
        a2 = a.reshape(nway, per)
        z = jnp.zeros((nway, perb - per), a.dtype)
        return jnp.concatenate([a2, z], axis=1).reshape(nway, perb // CH, CH)

    dst16 = shard(edge_index[1], NS)
    w16 = shard(edge_weight, NS)
    src_p = shard(edge_index[0], NW)
    dst_p = shard(edge_index[1], NW)
    w_p = shard(edge_weight, NW)
    x_p = jnp.pad(x, ((0, NP - N), (0, 0)))

    deg_parts = _deg_call(dst16, w16)
    g1, dinv_col = _tc1(deg_parts.T, x_p, W1)
    s1 = _agg_call(g1, src_p, dst_p, w_p)
    g2 = _tc2(s1, g1, dinv_col, b1.reshape(1, F_H), W2)
    s2 = _agg_call(g2, src_p, dst_p, w_p)
    out_col = _tc3(s2, g2, dinv_col, b2.reshape(1, F_H),
                   Wl, bl.reshape(1, 1))
    return out_col[:N, 0]
